# baseline (speedup 1.0000x reference)
import functools

import jax
import jax.numpy as jnp
from jax import lax
from jax.experimental import pallas as pl
from jax.experimental.pallas import tpu as pltpu
from jax.experimental.pallas import tpu_sc as plsc

NUM_KERNELS = 4
NUM_FEATURES = 8
GRID_SIZE = 128
NUM_POINTS = 1048576

NC = 2
NS = 16
NW = NC * NS
L = 16

C = 128
PER_W = NUM_POINTS // NW
NCHUNK = PER_W // C

_G = GRID_SIZE
_GG = _G * _G


def _sc_grid_gather():
    mesh = plsc.VectorSubcoreMesh(core_axis_name="c", subcore_axis_name="s")

    @functools.partial(
        pl.kernel,
        out_type=jax.ShapeDtypeStruct((NUM_POINTS * NUM_FEATURES,), jnp.float32),
        mesh=mesh,
        scratch_types=[
            pltpu.VMEM((C,), jnp.int32),
            pltpu.VMEM((3, C), jnp.float32),
            pltpu.VMEM((8, C), jnp.int32),
            pltpu.VMEM((8 * C, NUM_FEATURES), jnp.float32),
            pltpu.VMEM((3, C), jnp.float32),
            pltpu.VMEM((C * NUM_FEATURES,), jnp.float32),
            pltpu.SemaphoreType.DMA,
        ],
        compiler_params=pltpu.CompilerParams(
            needs_layout_passes=False, use_tc_tiling_on_sc=False
        ),
    )
    def body(idx_hbm, xs0_hbm, xs1_hbm, xs2_hbm, tab_hbm, out_hbm,
             ib, xb, gidx, gbuf, wb, ob, sem):
        xs_hbms = (xs0_hbm, xs1_hbm, xs2_hbm)
        wid = lax.axis_index("s") * NC + lax.axis_index("c")
        lane = lax.iota(jnp.int32, L)
        half = lane >> 3
        feat = lane & 7

        def chunk_body(ch, carry):
            cbase = pl.multiple_of(wid * PER_W + ch * C, C)
            pltpu.sync_copy(idx_hbm.at[pl.ds(cbase, C)], ib)
            for d in range(3):
                pltpu.sync_copy(xs_hbms[d].at[pl.ds(cbase, C)], xb.at[d])

            def phase_a(i, carry_a):
                sl = pl.ds(i * L, L)
                k = ib[sl]
                bs = []
                for d in range(3):
                    loc = xb[d, sl] * float(GRID_SIZE) - 0.5
                    t = loc.astype(jnp.int32)
                    b = jnp.minimum(jnp.maximum(t, 0), GRID_SIZE - 2)
                    w = jnp.clip(loc - b.astype(jnp.float32), 0.0, 1.0)
                    wb[d, sl] = w
                    bs.append(b)
                row0 = (((((k << 7) + bs[0]) << 7) + bs[1]) << 7) + bs[2]
                cidx = 0
                for dx in (0, 1):
                    for dy in (0, 1):
                        for dz in (0, 1):
                            off = dx * _GG + dy * _G + dz
                            gidx[cidx, sl] = row0 + off
                            cidx += 1
                return carry_a

            lax.fori_loop(0, C // L, phase_a, 0, unroll=2)

            copies = [
                pltpu.async_copy(
                    tab_hbm.at[gidx.at[c8]], gbuf.at[pl.ds(c8 * C, C)], sem
                )
                for c8 in range(8)
            ]
            for cp in copies:
                cp.wait()

            def phase_b(j, carry_b):
                grp = (j >> 3) * L
                p0 = j * 2
                dg = half + ((j & 7) * 2)
                sgl = pl.ds(grp, L)
                wxv = wb[0, sgl]
                wyv = wb[1, sgl]
                wzv = wb[2, sgl]
                wx = wxv.at[dg].get(mode="promise_in_bounds")
                wy = wyv.at[dg].get(mode="promise_in_bounds")
                wz = wzv.at[dg].get(mode="promise_in_bounds")

                pvec = half + p0

                def corner(c8):
                    v = plsc.load_gather(gbuf, [pvec + (c8 * C), feat])
                    s255 = 255.0 / (1.0 + jnp.exp(-v))
                    r = (s255 + 0.5).astype(jnp.int32)
                    return r.astype(jnp.float32) * (1.0 / 255.0)

                q000 = corner(0); q001 = corner(1)
                q010 = corner(2); q011 = corner(3)
                q100 = corner(4); q101 = corner(5)
                q110 = corner(6); q111 = corner(7)
                a00 = q000 + wz * (q001 - q000)
                a01 = q010 + wz * (q011 - q010)
                a10 = q100 + wz * (q101 - q100)
                a11 = q110 + wz * (q111 - q110)
                b0 = a00 + wy * (a01 - a00)
                b1 = a10 + wy * (a11 - a10)
                res = b0 + wx * (b1 - b0)
                ob[pl.ds(j * L, L)] = res
                return carry_b

            lax.fori_loop(0, C // 2, phase_b, 0, unroll=2)

            pltpu.sync_copy(
                ob, out_hbm.at[pl.ds(cbase * NUM_FEATURES, C * NUM_FEATURES)]
            )
            return carry

        lax.fori_loop(0, NCHUNK, chunk_body, 0)

    return body


_SC_KERNEL = _sc_grid_gather()


def kernel(idxs, xs, values_raw):
    idx_flat = idxs.reshape(NUM_POINTS)
    xs_t = xs.T
    table = values_raw.reshape(NUM_KERNELS * _G * _GG, NUM_FEATURES)
    out = _SC_KERNEL(idx_flat, xs_t[0], xs_t[1], xs_t[2], table)
    return out.reshape(NUM_POINTS, NUM_FEATURES)
